# baseline (speedup 1.0000x reference)
import functools

import jax
import jax.numpy as jnp
from jax.experimental import pallas as pl
from jax.experimental.pallas import tpu as pltpu

_H = 128


def _prologue_body(x3_ref, bg_ref, k_ref, acc_ref, *, nblocks):
    i = pl.program_id(0)
    blk = x3_ref[0]

    @pl.when(i == 0)
    def _():
        acc_ref[0] = 0

    acc_ref[0] += jnp.sum((blk[:, 2:3] != 0.0).astype(jnp.int32))
    bg_ref[...] = blk[:, 0:2]

    @pl.when(i == nblocks - 1)
    def _():
        k_ref[0] = acc_ref[0]


def _main_body(k_ref, si_ref, bg_ref, wt_ref, b_ref, out_ref, *, block_rows):
    i = pl.program_id(0)
    B = block_rows
    k = k_ref[0]
    v = si_ref[...].reshape(2 * B, _H)
    sir = jax.nn.sigmoid(
        jax.lax.dot_general(
            v, wt_ref[...], (((1,), (0,)), ((), ())),
            preferred_element_type=jnp.float32,
        )
        + b_ref[...]
    )
    s = sir[0:B]
    ii = sir[B:2 * B]
    row = i * B + jax.lax.broadcasted_iota(jnp.int32, (B, 1), 0)
    mask = (row < k).astype(jnp.float32)
    beta = bg_ref[:, 0:1]
    gamma = bg_ref[:, 1:2]
    ds = -beta * (ii * mask * s)
    dr = gamma * ii
    out_ref[0] = ds
    out_ref[1] = -ds - dr
    out_ref[2] = dr
    out_ref[3] = jnp.zeros_like(ds)


def kernel(t, x, W, b):
    del t
    n = x.shape[1]
    block_rows = 5000
    nb = n // block_rows
    wt = W.T
    b2 = b.reshape(1, _H)
    bg, karr = pl.pallas_call(
        functools.partial(_prologue_body, nblocks=nb),
        grid=(nb,),
        in_specs=[pl.BlockSpec((1, block_rows, _H), lambda i: (3, i, 0))],
        out_specs=[
            pl.BlockSpec((block_rows, 2), lambda i: (i, 0)),
            pl.BlockSpec(memory_space=pltpu.SMEM),
        ],
        out_shape=[
            jax.ShapeDtypeStruct((n, 2), jnp.float32),
            jax.ShapeDtypeStruct((1,), jnp.int32),
        ],
        scratch_shapes=[pltpu.SMEM((1,), jnp.int32)],
    )(x)
    out = pl.pallas_call(
        functools.partial(_main_body, block_rows=block_rows),
        grid=(nb,),
        in_specs=[
            pl.BlockSpec(memory_space=pltpu.SMEM),
            pl.BlockSpec((2, block_rows, _H), lambda i: (0, i, 0)),
            pl.BlockSpec((block_rows, 2), lambda i: (i, 0)),
            pl.BlockSpec((_H, _H), lambda i: (0, 0)),
            pl.BlockSpec((1, _H), lambda i: (0, 0)),
        ],
        out_specs=pl.BlockSpec((4, block_rows, _H), lambda i: (0, i, 0)),
        out_shape=jax.ShapeDtypeStruct((4, n, _H), jnp.float32),
    )(karr, x, bg, wt, b2)
    return out
